# Initial kernel scaffold; baseline (speedup 1.0000x reference)
import jax
import jax.numpy as jnp
from jax.experimental import pallas as pl


def kernel(x, mashup_embed, api_embed, domain_embed, edge_index, W_sde, b_sde, W_val, b_val, W_key, b_key, W_sie, b_sie):
    raise NotImplementedError("write your pallas kernel here")


# speedup vs baseline: 57.3425x; 57.3425x over previous
import functools

import jax
import jax.numpy as jnp
from jax import lax
from jax.experimental import pallas as pl
from jax.experimental.pallas import tpu as pltpu
from jax.experimental.pallas import tpu_sc as plsc

NUM_MASHUP = 8000
NUM_API = 2000
ATOT = NUM_MASHUP * NUM_API
FEAT = 128
CH = 768
BETA = 0.5
F32 = jnp.float32

SC_NC = 2
SC_NS = 16
SC_NW = SC_NC * SC_NS
LANES = 16
CHUNK = 128


def _cdiv(a, b):
    return (a + b - 1) // b


def _build_adjacency(src_half, dst_half, e2):
    n_chunks = _cdiv(e2, CHUNK)
    wch = _cdiv(n_chunks, SC_NW)
    nper = wch * CHUNK
    last_base = ((e2 - 1) // 8) * 8
    last_off = (e2 - 1) - last_base

    mesh = plsc.VectorSubcoreMesh(
        core_axis_name="c", subcore_axis_name="s",
        num_cores=SC_NC, num_subcores=SC_NS,
    )

    @functools.partial(
        pl.kernel,
        mesh=mesh,
        out_type=(),
        scratch_types=[
            pltpu.VMEM((nper,), jnp.int32),
            pltpu.VMEM((nper,), jnp.int32),
            pltpu.VMEM((LANES,), jnp.int32),
            pltpu.VMEM((LANES,), jnp.int32),
            pltpu.VMEM((nper,), jnp.int32),
            pltpu.VMEM((nper,), F32),
            pltpu.SemaphoreType.DMA,
        ],
    )
    def scat(src_hbm, dst_hbm, a_ref, src_v, dst_v, ls_v, ld_v, idx_v, ones_v, sem):
        wid = lax.axis_index("s") * SC_NC + lax.axis_index("c")
        base = wid * nper
        pltpu.sync_copy(src_hbm.at[pl.ds(base, nper)], src_v)
        pltpu.sync_copy(dst_hbm.at[pl.ds(base, nper)], dst_v)
        pltpu.sync_copy(src_hbm.at[pl.ds(last_base, LANES)], ls_v)
        pltpu.sync_copy(dst_hbm.at[pl.ds(last_base, LANES)], ld_v)
        ls = ls_v[...]
        ld = ld_v[...]
        last_flat = ls[last_off] * NUM_API + ld[last_off] - NUM_MASHUP

        def chunk_body(c, carry):
            for j in range(CHUNK // LANES):
                off = c * CHUNK + j * LANES
                s = src_v[pl.ds(off, LANES)]
                d = dst_v[pl.ds(off, LANES)]
                f = s * NUM_API + d - NUM_MASHUP
                gid = base + off + lax.iota(jnp.int32, LANES)
                f = jnp.where(gid < e2, f, last_flat)
                idx_v[pl.ds(off, LANES)] = f
                ones_v[pl.ds(off, LANES)] = jnp.full((LANES,), 1.0, F32)
            return carry

        lax.fori_loop(0, wch, chunk_body, 0)
        pltpu.async_copy(ones_v, a_ref.at[idx_v], sem).wait()

    a0 = jnp.zeros((ATOT,), F32)
    aref = jax.new_ref(a0)
    scat(src_half, dst_half, aref)
    return aref[...]


def _attention(x, dom, w_sde, b_sde, w_val, b_val, w_key, b_key):
    def body(x_r, dom_r, wsde_r, bsde_r, wval_r, bval_r, wkey_r, bkey_r, z_r):
        v_mi = jax.nn.sigmoid(
            jnp.dot(x_r[...], wsde_r[...], preferred_element_type=F32) + bsde_r[...]
        )
        v_val = jax.nn.sigmoid(
            jnp.dot(dom_r[...], wval_r[...], preferred_element_type=F32) + bval_r[...]
        )
        v_key = jax.nn.sigmoid(
            jnp.dot(dom_r[...], wkey_r[...], preferred_element_type=F32) + bkey_r[...]
        )
        al = lax.dot_general(
            v_mi, v_key, (((1,), (1,)), ((), ())), preferred_element_type=F32
        )
        alpha = al / jnp.sum(al, axis=1, keepdims=True)
        s_m = jnp.dot(alpha, v_val, preferred_element_type=F32)
        z_r[...] = (1.0 - BETA) * s_m + BETA * v_mi

    b = x.shape[0]
    return pl.pallas_call(
        body, out_shape=jax.ShapeDtypeStruct((b, FEAT), F32)
    )(x, dom, w_sde, b_sde, w_val, b_val, w_key, b_key)


def _proj(inp, w, b2d, blk):
    n = inp.shape[0]
    g = n // blk

    def body(i_r, w_r, b_r, o_r):
        o_r[...] = jax.nn.sigmoid(
            jnp.dot(i_r[...], w_r[...], preferred_element_type=F32) + b_r[...]
        )

    return pl.pallas_call(
        body,
        grid=(g,),
        in_specs=[
            pl.BlockSpec((blk, CH), lambda i: (i, 0)),
            pl.BlockSpec((CH, FEAT), lambda i: (0, 0)),
            pl.BlockSpec((1, FEAT), lambda i: (0, 0)),
        ],
        out_specs=pl.BlockSpec((blk, FEAT), lambda i: (i, 0)),
        out_shape=jax.ShapeDtypeStruct((n, FEAT), F32),
    )(inp, w, b2d)


RBLK = 1000
NSTEP = NUM_MASHUP // RBLK


def _rsqrtz(x):
    return jnp.where(x > 0, lax.rsqrt(x), 0.0)


def _gcn1(a, v_m, v_s):
    def body(a_r, vm_r, vs_r, w_r, dega_r, accu_r, accda_r):
        i = pl.program_id(0)
        ab = a_r[...]
        degm = jnp.sum(ab, axis=1, keepdims=True)
        dism = _rsqrtz(degm)
        contrib_u = lax.dot_general(
            ab, dism * vm_r[...], (((0,), (0,)), ((), ())), preferred_element_type=F32
        )
        ones8 = jnp.ones((RBLK, 8), F32)
        contrib_da = lax.dot_general(
            ab, ones8, (((0,), (0,)), ((), ())), preferred_element_type=F32
        )

        @pl.when(i == 0)
        def _():
            accu_r[...] = contrib_u
            accda_r[...] = contrib_da

        @pl.when(i > 0)
        def _():
            accu_r[...] += contrib_u
            accda_r[...] += contrib_da

        @pl.when(i == NSTEP - 1)
        def _():
            dega = accda_r[...]
            disa = _rsqrtz(dega[:, 0:1])
            w_r[...] = vs_r[...] + disa * accu_r[...]
            dega_r[...] = dega

    return pl.pallas_call(
        body,
        grid=(NSTEP,),
        in_specs=[
            pl.BlockSpec((RBLK, NUM_API), lambda i: (i, 0)),
            pl.BlockSpec((RBLK, FEAT), lambda i: (i, 0)),
            pl.BlockSpec((NUM_API, FEAT), lambda i: (0, 0)),
        ],
        out_specs=[
            pl.BlockSpec((NUM_API, FEAT), lambda i: (0, 0)),
            pl.BlockSpec((NUM_API, 8), lambda i: (0, 0)),
        ],
        out_shape=[
            jax.ShapeDtypeStruct((NUM_API, FEAT), F32),
            jax.ShapeDtypeStruct((NUM_API, 8), F32),
        ],
        scratch_shapes=[
            pltpu.VMEM((NUM_API, FEAT), F32),
            pltpu.VMEM((NUM_API, 8), F32),
        ],
    )(a, v_m, v_s)


def _gcn2(a, w, dega, z_m):
    b = z_m.shape[0]

    def body(a_r, w_r, dega_r, zm_r, pred_r, acc2_r):
        i = pl.program_id(0)
        ab = a_r[...]
        disa = _rsqrtz(dega_r[:, 0:1])
        wa = disa * w_r[...]
        traw = lax.dot_general(
            ab, wa, (((1,), (0,)), ((), ())), preferred_element_type=F32
        )
        degm = jnp.sum(ab, axis=1, keepdims=True)
        dm2 = jnp.where(degm > 0, 1.0 / degm, 0.0)
        contrib = lax.dot_general(
            ab, dm2 * traw, (((0,), (0,)), ((), ())), preferred_element_type=F32
        )

        @pl.when(i == 0)
        def _():
            acc2_r[...] = contrib

        @pl.when(i > 0)
        def _():
            acc2_r[...] += contrib

        @pl.when(i == NSTEP - 1)
        def _():
            out_s = 0.25 * (w_r[...] + disa * acc2_r[...])
            pred_r[...] = lax.dot_general(
                zm_r[...], out_s, (((1,), (1,)), ((), ())), preferred_element_type=F32
            )

    return pl.pallas_call(
        body,
        grid=(NSTEP,),
        in_specs=[
            pl.BlockSpec((RBLK, NUM_API), lambda i: (i, 0)),
            pl.BlockSpec((NUM_API, FEAT), lambda i: (0, 0)),
            pl.BlockSpec((NUM_API, 8), lambda i: (0, 0)),
            pl.BlockSpec((b, FEAT), lambda i: (0, 0)),
        ],
        out_specs=pl.BlockSpec((b, NUM_API), lambda i: (0, 0)),
        out_shape=jax.ShapeDtypeStruct((b, NUM_API), F32),
        scratch_shapes=[pltpu.VMEM((NUM_API, FEAT), F32)],
    )(a, w, dega, z_m)


def kernel(x, mashup_embed, api_embed, domain_embed, edge_index,
           W_sde, b_sde, W_val, b_val, W_key, b_key, W_sie, b_sie):
    e = edge_index.shape[1]
    e2 = e // 2
    src_half = edge_index[0]
    dst_half = edge_index[1]

    a_flat = _build_adjacency(src_half, dst_half, e2)
    a = a_flat.reshape(NUM_MASHUP, NUM_API)

    b_sde2 = b_sde.reshape(1, FEAT)
    b_val2 = b_val.reshape(1, FEAT)
    b_key2 = b_key.reshape(1, FEAT)
    b_sie2 = b_sie.reshape(1, FEAT)

    z_m = _attention(x, domain_embed, W_sde, b_sde2, W_val, b_val2, W_key, b_key2)
    v_m = _proj(mashup_embed, W_sde, b_sde2, 1000)
    v_s = _proj(api_embed, W_sie, b_sie2, 1000)

    w, dega = _gcn1(a, v_m, v_s)
    pred = _gcn2(a, w, dega, z_m)
    return pred


# speedup vs baseline: 57.3705x; 1.0005x over previous
import functools

import jax
import jax.numpy as jnp
from jax import lax
from jax.experimental import pallas as pl
from jax.experimental.pallas import tpu as pltpu
from jax.experimental.pallas import tpu_sc as plsc

NUM_MASHUP = 8000
NUM_API = 2000
ATOT = NUM_MASHUP * NUM_API
FEAT = 128
CH = 768
BETA = 0.5
F32 = jnp.float32

SC_NC = 2
SC_NS = 16
SC_NW = SC_NC * SC_NS
LANES = 16
CHUNK = 128


def _cdiv(a, b):
    return (a + b - 1) // b


def _build_adjacency(src_half, dst_half, e2):
    n_chunks = _cdiv(e2, CHUNK)
    wch = _cdiv(n_chunks, SC_NW)
    nper = wch * CHUNK
    last_base = ((e2 - 1) // 8) * 8
    last_off = (e2 - 1) - last_base

    mesh = plsc.VectorSubcoreMesh(
        core_axis_name="c", subcore_axis_name="s",
        num_cores=SC_NC, num_subcores=SC_NS,
    )

    nsplit = 8
    slen = nper // nsplit
    glen = slen // LANES

    @functools.partial(
        pl.kernel,
        mesh=mesh,
        out_type=(),
        scratch_types=[
            pltpu.VMEM((nper,), jnp.int32),
            pltpu.VMEM((nper,), jnp.int32),
            pltpu.VMEM((LANES,), jnp.int32),
            pltpu.VMEM((LANES,), jnp.int32),
            pltpu.VMEM((slen,), F32),
            pltpu.SemaphoreType.DMA,
        ]
        + [pltpu.VMEM((slen,), jnp.int32) for _ in range(nsplit)],
    )
    def scat(src_hbm, dst_hbm, a_ref, src_v, dst_v, ls_v, ld_v, ones_v, sem,
             *idx_refs):
        wid = lax.axis_index("s") * SC_NC + lax.axis_index("c")
        base = wid * nper
        pltpu.sync_copy(src_hbm.at[pl.ds(base, nper)], src_v)
        pltpu.sync_copy(dst_hbm.at[pl.ds(base, nper)], dst_v)
        pltpu.sync_copy(src_hbm.at[pl.ds(last_base, LANES)], ls_v)
        pltpu.sync_copy(dst_hbm.at[pl.ds(last_base, LANES)], ld_v)
        ls = ls_v[...]
        ld = ld_v[...]
        last_flat = ls[last_off] * NUM_API + ld[last_off] - NUM_MASHUP

        def fill_ones(g, carry):
            ones_v[pl.ds(g * LANES, LANES)] = jnp.full((LANES,), 1.0, F32)
            return carry

        lax.fori_loop(0, glen, fill_ones, 0)

        for k in range(nsplit):
            def grp_body(g, carry, _k=k):
                off = _k * slen + g * LANES
                s = src_v[pl.ds(off, LANES)]
                d = dst_v[pl.ds(off, LANES)]
                f = s * NUM_API + d - NUM_MASHUP
                gid = base + off + lax.iota(jnp.int32, LANES)
                f = jnp.where(gid < e2, f, last_flat)
                idx_refs[_k][pl.ds(g * LANES, LANES)] = f
                return carry

            lax.fori_loop(0, glen, grp_body, 0)

        copies = [
            pltpu.async_copy(ones_v, a_ref.at[idx_refs[k]], sem)
            for k in range(nsplit)
        ]
        for c in copies:
            c.wait()

    a0 = jnp.zeros((ATOT,), F32)
    aref = jax.new_ref(a0)
    scat(src_half, dst_half, aref)
    return aref[...]


def _attention(x, dom, w_sde, b_sde, w_val, b_val, w_key, b_key):
    def body(x_r, dom_r, wsde_r, bsde_r, wval_r, bval_r, wkey_r, bkey_r, z_r):
        v_mi = jax.nn.sigmoid(
            jnp.dot(x_r[...], wsde_r[...], preferred_element_type=F32) + bsde_r[...]
        )
        v_val = jax.nn.sigmoid(
            jnp.dot(dom_r[...], wval_r[...], preferred_element_type=F32) + bval_r[...]
        )
        v_key = jax.nn.sigmoid(
            jnp.dot(dom_r[...], wkey_r[...], preferred_element_type=F32) + bkey_r[...]
        )
        al = lax.dot_general(
            v_mi, v_key, (((1,), (1,)), ((), ())), preferred_element_type=F32
        )
        alpha = al / jnp.sum(al, axis=1, keepdims=True)
        s_m = jnp.dot(alpha, v_val, preferred_element_type=F32)
        z_r[...] = (1.0 - BETA) * s_m + BETA * v_mi

    b = x.shape[0]
    return pl.pallas_call(
        body, out_shape=jax.ShapeDtypeStruct((b, FEAT), F32)
    )(x, dom, w_sde, b_sde, w_val, b_val, w_key, b_key)


def _proj(inp, w, b2d, blk):
    n = inp.shape[0]
    g = n // blk

    def body(i_r, w_r, b_r, o_r):
        o_r[...] = jax.nn.sigmoid(
            jnp.dot(i_r[...], w_r[...], preferred_element_type=F32) + b_r[...]
        )

    return pl.pallas_call(
        body,
        grid=(g,),
        in_specs=[
            pl.BlockSpec((blk, CH), lambda i: (i, 0)),
            pl.BlockSpec((CH, FEAT), lambda i: (0, 0)),
            pl.BlockSpec((1, FEAT), lambda i: (0, 0)),
        ],
        out_specs=pl.BlockSpec((blk, FEAT), lambda i: (i, 0)),
        out_shape=jax.ShapeDtypeStruct((n, FEAT), F32),
    )(inp, w, b2d)


RBLK = 1000
NSTEP = NUM_MASHUP // RBLK


def _rsqrtz(x):
    return jnp.where(x > 0, lax.rsqrt(x), 0.0)


def _gcn1(a, v_m, v_s):
    def body(a_r, vm_r, vs_r, w_r, dega_r, accu_r, accda_r):
        i = pl.program_id(0)
        ab = a_r[...]
        degm = jnp.sum(ab, axis=1, keepdims=True)
        dism = _rsqrtz(degm)
        contrib_u = lax.dot_general(
            ab, dism * vm_r[...], (((0,), (0,)), ((), ())), preferred_element_type=F32
        )
        ones8 = jnp.ones((RBLK, 8), F32)
        contrib_da = lax.dot_general(
            ab, ones8, (((0,), (0,)), ((), ())), preferred_element_type=F32
        )

        @pl.when(i == 0)
        def _():
            accu_r[...] = contrib_u
            accda_r[...] = contrib_da

        @pl.when(i > 0)
        def _():
            accu_r[...] += contrib_u
            accda_r[...] += contrib_da

        @pl.when(i == NSTEP - 1)
        def _():
            dega = accda_r[...]
            disa = _rsqrtz(dega[:, 0:1])
            w_r[...] = vs_r[...] + disa * accu_r[...]
            dega_r[...] = dega

    return pl.pallas_call(
        body,
        grid=(NSTEP,),
        in_specs=[
            pl.BlockSpec((RBLK, NUM_API), lambda i: (i, 0)),
            pl.BlockSpec((RBLK, FEAT), lambda i: (i, 0)),
            pl.BlockSpec((NUM_API, FEAT), lambda i: (0, 0)),
        ],
        out_specs=[
            pl.BlockSpec((NUM_API, FEAT), lambda i: (0, 0)),
            pl.BlockSpec((NUM_API, 8), lambda i: (0, 0)),
        ],
        out_shape=[
            jax.ShapeDtypeStruct((NUM_API, FEAT), F32),
            jax.ShapeDtypeStruct((NUM_API, 8), F32),
        ],
        scratch_shapes=[
            pltpu.VMEM((NUM_API, FEAT), F32),
            pltpu.VMEM((NUM_API, 8), F32),
        ],
    )(a, v_m, v_s)


def _gcn2(a, w, dega, z_m):
    b = z_m.shape[0]

    def body(a_r, w_r, dega_r, zm_r, pred_r, acc2_r):
        i = pl.program_id(0)
        ab = a_r[...]
        disa = _rsqrtz(dega_r[:, 0:1])
        wa = disa * w_r[...]
        traw = lax.dot_general(
            ab, wa, (((1,), (0,)), ((), ())), preferred_element_type=F32
        )
        degm = jnp.sum(ab, axis=1, keepdims=True)
        dm2 = jnp.where(degm > 0, 1.0 / degm, 0.0)
        contrib = lax.dot_general(
            ab, dm2 * traw, (((0,), (0,)), ((), ())), preferred_element_type=F32
        )

        @pl.when(i == 0)
        def _():
            acc2_r[...] = contrib

        @pl.when(i > 0)
        def _():
            acc2_r[...] += contrib

        @pl.when(i == NSTEP - 1)
        def _():
            out_s = 0.25 * (w_r[...] + disa * acc2_r[...])
            pred_r[...] = lax.dot_general(
                zm_r[...], out_s, (((1,), (1,)), ((), ())), preferred_element_type=F32
            )

    return pl.pallas_call(
        body,
        grid=(NSTEP,),
        in_specs=[
            pl.BlockSpec((RBLK, NUM_API), lambda i: (i, 0)),
            pl.BlockSpec((NUM_API, FEAT), lambda i: (0, 0)),
            pl.BlockSpec((NUM_API, 8), lambda i: (0, 0)),
            pl.BlockSpec((b, FEAT), lambda i: (0, 0)),
        ],
        out_specs=pl.BlockSpec((b, NUM_API), lambda i: (0, 0)),
        out_shape=jax.ShapeDtypeStruct((b, NUM_API), F32),
        scratch_shapes=[pltpu.VMEM((NUM_API, FEAT), F32)],
    )(a, w, dega, z_m)


def kernel(x, mashup_embed, api_embed, domain_embed, edge_index,
           W_sde, b_sde, W_val, b_val, W_key, b_key, W_sie, b_sie):
    e = edge_index.shape[1]
    e2 = e // 2
    src_half = edge_index[0]
    dst_half = edge_index[1]

    a_flat = _build_adjacency(src_half, dst_half, e2)
    a = a_flat.reshape(NUM_MASHUP, NUM_API)

    b_sde2 = b_sde.reshape(1, FEAT)
    b_val2 = b_val.reshape(1, FEAT)
    b_key2 = b_key.reshape(1, FEAT)
    b_sie2 = b_sie.reshape(1, FEAT)

    z_m = _attention(x, domain_embed, W_sde, b_sde2, W_val, b_val2, W_key, b_key2)
    v_m = _proj(mashup_embed, W_sde, b_sde2, 1000)
    v_s = _proj(api_embed, W_sie, b_sie2, 1000)

    w, dega = _gcn1(a, v_m, v_s)
    pred = _gcn2(a, w, dega, z_m)
    return pred
